# baseline (speedup 1.0000x reference)
import functools

import jax
import jax.numpy as jnp
from jax import lax
from jax.experimental import pallas as pl
from jax.experimental.pallas import tpu as pltpu
from jax.experimental.pallas import tpu_sc as plsc

_NUM_FIELDS = 26
_PER_FIELD_VOCAB = 100000
_EMBED_DIM = 32
_BATCH = 16384

_NC = 2
_NS = 16
_L = 16
_NW = _NC * _NS
_BW = _BATCH // _NW
_ROWS = 2600000
_QROWS = 650000
_W = 256
_NSLAB = 2599936 // _W
_TAIL0 = _NSLAB * _W
_CH = 128

_mesh = plsc.VectorSubcoreMesh(core_axis_name="c", subcore_axis_name="s")


def _transpose_slab(vin, vout, width, nrows):
    e_lo = lax.iota(jnp.int32, _L)
    e_hi = e_lo + _L

    def row8_body(r8, c):
        for k in range(8):
            r = r8 * 8 + k
            v0 = 4 * r
            for g in range(8):
                vloc = jnp.full((_L,), v0 + g // 2, jnp.int32)
                vout[r, pl.ds(g * _L, _L)] = plsc.load_gather(
                    vin, [e_hi if g % 2 else e_lo, vloc]
                )
        return c

    lax.fori_loop(0, nrows // 8, row8_body, 0)


@functools.partial(
    pl.kernel,
    out_type=jax.ShapeDtypeStruct((_QROWS, 128), jnp.float32),
    mesh=_mesh,
    compiler_params=pltpu.CompilerParams(needs_layout_passes=False),
    scratch_types=[
        pltpu.VMEM((_EMBED_DIM, _W), jnp.float32),
        pltpu.VMEM((_EMBED_DIM, _W), jnp.float32),
        pltpu.VMEM((_W // 4, 128), jnp.float32),
        pltpu.VMEM((_W // 4, 128), jnp.float32),
        pltpu.SemaphoreType.DMA,
        pltpu.SemaphoreType.DMA,
    ],
)
def _to_superrows(tT_hbm, tailq_hbm, tq_hbm, vin0, vin1, vout0, vout1, sem0, sem1):
    wid = lax.axis_index("s") * _NC + lax.axis_index("c")
    bufs = ((vin0, vout0, sem0), (vin1, vout1, sem1))

    def pair_body(i2, carry):
        sl0 = wid + _NW * (2 * i2)
        sl1 = wid + _NW * (2 * i2 + 1)
        both = sl1 < _NSLAB

        @pl.when(both)
        def _():
            c0 = pl.multiple_of(sl0 * _W, 128)
            c1 = pl.multiple_of(sl1 * _W, 128)
            h0 = pltpu.async_copy(tT_hbm.at[:, pl.ds(c0, _W)], vin0, sem0)
            h1 = pltpu.async_copy(tT_hbm.at[:, pl.ds(c1, _W)], vin1, sem1)
            h0.wait()
            _transpose_slab(vin0, vout0, _W, _W // 4)
            r0 = pl.multiple_of(sl0 * (_W // 4), 8)
            o0 = pltpu.async_copy(vout0, tq_hbm.at[pl.ds(r0, _W // 4)], sem0)
            h1.wait()
            _transpose_slab(vin1, vout1, _W, _W // 4)
            r1 = pl.multiple_of(sl1 * (_W // 4), 8)
            o1 = pltpu.async_copy(vout1, tq_hbm.at[pl.ds(r1, _W // 4)], sem1)
            o0.wait()
            o1.wait()

        @pl.when(jnp.logical_and(sl0 < _NSLAB, jnp.logical_not(both)))
        def _():
            c0 = pl.multiple_of(sl0 * _W, 128)
            pltpu.async_copy(tT_hbm.at[:, pl.ds(c0, _W)], vin0, sem0).wait()
            _transpose_slab(vin0, vout0, _W, _W // 4)
            r0 = pl.multiple_of(sl0 * (_W // 4), 8)
            pltpu.async_copy(vout0, tq_hbm.at[pl.ds(r0, _W // 4)], sem0).wait()

        return carry

    lax.fori_loop(0, (_NSLAB + 2 * _NW - 1) // (2 * _NW), pair_body, 0)

    @pl.when(wid == _NW - 1)
    def _():
        pltpu.async_copy(tailq_hbm, vout0.at[pl.ds(0, 16)], sem0).wait()
        pltpu.async_copy(
            vout0.at[pl.ds(0, 16)], tq_hbm.at[pl.ds(_TAIL0 // 4, 16)], sem0
        ).wait()


@functools.partial(
    pl.kernel,
    out_type=jax.ShapeDtypeStruct((_NUM_FIELDS, _EMBED_DIM, _BATCH), jnp.float32),
    mesh=_mesh,
    compiler_params=pltpu.CompilerParams(needs_layout_passes=False),
    scratch_types=[
        pltpu.VMEM((_NUM_FIELDS, _BW), jnp.int32),
        pltpu.VMEM((_CH,), jnp.int32),
        pltpu.VMEM((_CH,), jnp.int32),
        pltpu.VMEM((_CH, 128), jnp.float32),
        pltpu.VMEM((_EMBED_DIM, _BW), jnp.float32),
        pltpu.SemaphoreType.DMA,
    ],
)
def _gather(xT_hbm, tq_hbm, out_hbm, xb_v, q_v, s_v, rows_v, out_v, sem):
    wid = lax.axis_index("s") * _NC + lax.axis_index("c")
    b0 = pl.multiple_of(wid * _BW, 128)
    pltpu.sync_copy(xT_hbm.at[:, pl.ds(b0, _BW)], xb_v)

    def field_body(f, carry):
        off = f * _PER_FIELD_VOCAB

        def chunk_body(q, c):
            def idx_grp(g, c2):
                v = xb_v[f, pl.ds(q * _CH + g * _L, _L)] + off
                q_v[pl.ds(g * _L, _L)] = lax.shift_right_logical(v, 2)
                s_v[pl.ds(g * _L, _L)] = lax.shift_left(jnp.bitwise_and(v, 3), 5)
                return c2

            lax.fori_loop(0, _CH // _L, idx_grp, 0)
            pltpu.async_copy(tq_hbm.at[q_v], rows_v, sem).wait()

            def ext_grp(g, c2):
                jvec = g * _L + lax.iota(jnp.int32, _L)
                svec = s_v[pl.ds(g * _L, _L)]
                for e in range(_EMBED_DIM):
                    out_v[e, pl.ds(q * _CH + g * _L, _L)] = plsc.load_gather(
                        rows_v, [jvec, svec + e]
                    )
                return c2

            lax.fori_loop(0, _CH // _L, ext_grp, 0)
            return c

        lax.fori_loop(0, _BW // _CH, chunk_body, 0)
        pltpu.sync_copy(out_v, out_hbm.at[f, :, pl.ds(b0, _BW)])
        return carry

    lax.fori_loop(0, _NUM_FIELDS, field_body, 0)


@jax.jit
def kernel(x, embedding_table):
    tailq = embedding_table[_TAIL0:].reshape(16, 128)
    tq = _to_superrows(embedding_table.T, tailq)
    out = _gather(x.T, tq)
    return jnp.transpose(out, (2, 0, 1))


# speedup vs baseline: 1.8706x; 1.8706x over previous
import functools

import jax
import jax.numpy as jnp
from jax import lax
from jax.experimental import pallas as pl
from jax.experimental.pallas import tpu as pltpu
from jax.experimental.pallas import tpu_sc as plsc

_NUM_FIELDS = 26
_PER_FIELD_VOCAB = 100000
_EMBED_DIM = 32
_BATCH = 16384

_NC = 2
_NS = 16
_L = 16
_NW = _NC * _NS
_BW = _BATCH // _NW
_NGRP = 325000
_CH = 64

_mesh = plsc.VectorSubcoreMesh(core_axis_name="c", subcore_axis_name="s")


@functools.partial(
    pl.kernel,
    out_type=jax.ShapeDtypeStruct((_NUM_FIELDS, _EMBED_DIM, _BATCH), jnp.float32),
    mesh=_mesh,
    compiler_params=pltpu.CompilerParams(needs_layout_passes=False),
    scratch_types=[
        pltpu.VMEM((_BW,), jnp.int32),
        pltpu.VMEM((_CH,), jnp.int32),
        pltpu.VMEM((_CH, 8, _EMBED_DIM), jnp.float32),
        pltpu.VMEM((_EMBED_DIM, _BW), jnp.float32),
        pltpu.SemaphoreType.DMA,
    ],
)
def _emb_lookup(xT_hbm, tg_hbm, out_hbm, xb_v, s_v, rows_v, out_v, sem):
    wid = lax.axis_index("s") * _NC + lax.axis_index("c")
    b0 = pl.multiple_of(wid * _BW, 128)

    def field_body(f, carry):
        off = f * _PER_FIELD_VOCAB
        pltpu.sync_copy(xT_hbm.at[f, pl.ds(b0, _BW)], xb_v)

        def chunk_body(q, c):
            copies = []
            for g in range(_CH // _L):
                vv = xb_v[pl.ds(q * _CH + g * _L, _L)] + off
                s_v[pl.ds(g * _L, _L)] = jnp.bitwise_and(vv, 7)
                for j in range(_L):
                    grp = lax.shift_right_logical(vv[j], 3)
                    copies.append(
                        pltpu.async_copy(
                            tg_hbm.at[pl.ds(grp, 1)],
                            rows_v.at[pl.ds(g * _L + j, 1)],
                            sem,
                        )
                    )
            for cp in copies:
                cp.wait()

            def ext_grp(g, c2):
                jvec = g * _L + lax.iota(jnp.int32, _L)
                svec = s_v[pl.ds(g * _L, _L)]
                for e in range(_EMBED_DIM):
                    evec = jnp.full((_L,), e, jnp.int32)
                    out_v[e, pl.ds(q * _CH + g * _L, _L)] = plsc.load_gather(
                        rows_v, [jvec, svec, evec]
                    )
                return c2

            lax.fori_loop(0, _CH // _L, ext_grp, 0)
            return c

        lax.fori_loop(0, _BW // _CH, chunk_body, 0)
        pltpu.sync_copy(out_v, out_hbm.at[f, :, pl.ds(b0, _BW)])
        return carry

    lax.fori_loop(0, _NUM_FIELDS, field_body, 0)


@jax.jit
def kernel(x, embedding_table):
    out = _emb_lookup(x.T, embedding_table.reshape(_NGRP, 8, _EMBED_DIM))
    return jnp.transpose(out, (2, 0, 1))
